# baseline (speedup 1.0000x reference)
import functools

import jax
import jax.numpy as jnp
from jax import lax
from jax.experimental import pallas as pl
from jax.experimental.pallas import tpu as pltpu
from jax.experimental.pallas import tpu_sc as plsc

N = 50000
E = 800000
D = 64
Y = 64
HALF = 32

NPAD = 50176
NP4 = NPAD // 4
BLKP = 256
NBP = NP4 // BLKP
BLKA = 64
NBA = NP4 // BLKA
PK = 4 * D

CHUNK = 128
NCH = E // CHUNK
NTILES = 16
RING = 6
NGRP = -(-((NCH // NTILES) + 1) // RING)
ROWS_PER_TILE = NPAD // NTILES
GROWS = RING * CHUNK

_EPS = 1e-12


def _spmm_body(rc, cur_a, cur_b, pool_a, pool_b,
               accum, ibuf, gbuf, isem, gsem, ssem):
    c = lax.axis_index("c")
    s = lax.axis_index("s")
    cnt = 390 + jnp.where(s < 10, 1, 0)
    start = s * 390 + jnp.minimum(s, 10)
    base = s * ROWS_PER_TILE

    def run_half(cur_hbm, pool_hbm):
        z16 = jnp.zeros((16,), jnp.float32)

        def zbody(i, carry):
            gbuf[i, 0:16] = z16
            gbuf[i, 16:32] = z16
            return carry

        lax.fori_loop(0, GROWS, zbody, 0)
        for t in range(ROWS_PER_TILE // GROWS):
            pltpu.sync_copy(gbuf, accum.at[pl.ds(base + t * GROWS, GROWS)])
        rem = ROWS_PER_TILE % GROWS
        pltpu.sync_copy(gbuf.at[pl.ds(0, rem)],
                        accum.at[pl.ds(base + (ROWS_PER_TILE // GROWS) * GROWS,
                                       rem)])
        plsc.subcore_barrier()

        for b in range(RING):
            pltpu.async_copy(rc.at[start + b], ibuf.at[0, b], isem)

        def body(g, carry):
            par = lax.rem(g, 2)
            nxt = 1 - par

            @pl.when(g > 0)
            def _():
                for b in range(RING):
                    pltpu.make_async_copy(
                        gbuf.at[pl.ds(b * CHUNK, CHUNK)],
                        accum.at[ibuf.at[nxt, b, 0]], ssem).wait()

            for b in range(RING):
                @pl.when(g * RING + b < cnt)
                def _(b=b):
                    pltpu.make_async_copy(rc.at[0], ibuf.at[par, b],
                                          isem).wait()
                    pltpu.async_copy(cur_hbm.at[ibuf.at[par, b, 1]],
                                     gbuf.at[pl.ds(b * CHUNK, CHUNK)], gsem)

            for b in range(RING):
                @pl.when((g + 1) * RING + b < cnt)
                def _(b=b):
                    pltpu.async_copy(rc.at[start + (g + 1) * RING + b],
                                     ibuf.at[nxt, b], isem)

            for b in range(RING):
                @pl.when(g * RING + b < cnt)
                def _(b=b):
                    pltpu.make_async_copy(
                        cur_hbm.at[ibuf.at[par, b, 1]],
                        gbuf.at[pl.ds(b * CHUNK, CHUNK)], gsem).wait()
                    pltpu.async_copy(gbuf.at[pl.ds(b * CHUNK, CHUNK)],
                                     accum.at[ibuf.at[par, b, 0]], ssem,
                                     add=True)
            return carry

        lax.fori_loop(0, NGRP, body, 0)

        lastpar = (NGRP - 1) % 2
        for b in range(RING):
            @pl.when((NGRP - 1) * RING + b < cnt)
            def _(b=b):
                pltpu.make_async_copy(
                    gbuf.at[pl.ds(b * CHUNK, CHUNK)],
                    accum.at[ibuf.at[lastpar, b, 0]], ssem).wait()

        plsc.subcore_barrier()
        pltpu.sync_copy(accum.at[pl.ds(base, ROWS_PER_TILE)],
                        pool_hbm.at[pl.ds(base, ROWS_PER_TILE)])

    @pl.when(c == 0)
    def _():
        run_half(cur_a, pool_a)

    @pl.when(c == 1)
    def _():
        run_half(cur_b, pool_b)


@functools.cache
def _get_spmm():
    return pl.kernel(
        _spmm_body,
        out_type=[jax.ShapeDtypeStruct((NPAD, HALF), jnp.float32),
                  jax.ShapeDtypeStruct((NPAD, HALF), jnp.float32)],
        mesh=plsc.VectorSubcoreMesh(core_axis_name="c", subcore_axis_name="s",
                                    num_cores=2, num_subcores=NTILES),
        scratch_types=[
            pltpu.VMEM_SHARED((NPAD, HALF), jnp.float32),
            pltpu.VMEM((2, RING, 2, CHUNK), jnp.int32),
            pltpu.VMEM((GROWS, HALF), jnp.float32),
            pltpu.SemaphoreType.DMA,
            pltpu.SemaphoreType.DMA,
            pltpu.SemaphoreType.DMA,
        ],
        compiler_params=pltpu.CompilerParams(use_tc_tiling_on_sc=False),
    )


def _bd4(w, r, c):
    t4 = jnp.concatenate([jnp.concatenate([w] * 4, axis=1)] * 4, axis=0)
    ri = lax.broadcasted_iota(jnp.int32, (4 * r, 4 * c), 0)
    ci = lax.broadcasted_iota(jnp.int32, (4 * r, 4 * c), 1)
    return jnp.where(ri // r == ci // c, t4, 0.0)


def _l2n(x):
    n = jnp.sqrt(jnp.sum(x * x, axis=1, keepdims=True))
    return x / jnp.maximum(n, _EPS)


def _l2n_pk(x, jd):
    nsq = jnp.dot(x * x, jd, preferred_element_type=jnp.float32)
    return x / jnp.maximum(jnp.sqrt(nsq), _EPS)


def _weights_body(pc, pc2, pc3, wn, ew, eb,
                  w1a_pk, w1b_pk, w2a_pk, w2b_pk, enc_pk, b_pk,
                  sel_a, sel_b, jd, w1o, w2o, y0, pcat, szcat):
    w1 = jnp.dot(pc[...], pc3[0:D, :], preferred_element_type=jnp.float32)
    w2 = jnp.dot(pc2[...], pc3[D:2 * D, :],
                 preferred_element_type=jnp.float32)
    w1o[...] = w1
    w2o[...] = w2
    w1a_pk[...] = _bd4(w1[0:HALF, :], HALF, D)
    w1b_pk[...] = _bd4(w1[HALF:D, :], HALF, D)
    w2a_pk[...] = _bd4(w2[0:HALF, :], HALF, D)
    w2b_pk[...] = _bd4(w2[HALF:D, :], HALF, D)
    enc_pk[...] = _bd4(ew[...], 2, D)
    b_pk[...] = jnp.concatenate([eb[...]] * 4, axis=1)
    ri = lax.broadcasted_iota(jnp.int32, (PK, 2 * D), 0)
    ci = lax.broadcasted_iota(jnp.int32, (PK, 2 * D), 1)
    hit = ri // D == ci // HALF
    sel_a[...] = jnp.where(hit & (ri % D == ci % HALF), 1.0, 0.0)
    sel_b[...] = jnp.where(hit & (ri % D == ci % HALF + HALF), 1.0, 0.0)
    ri2 = lax.broadcasted_iota(jnp.int32, (PK, PK), 0)
    ci2 = lax.broadcasted_iota(jnp.int32, (PK, PK), 1)
    jd[...] = jnp.where(ri2 // D == ci2 // D, 1.0, 0.0)
    r = jnp.maximum(wn[0:1, :] + wn[1:2, :], 0.0)
    y0[...] = _l2n(jnp.broadcast_to(r, (Y, D)))
    tt = lax.broadcasted_iota(jnp.int32, (4, 4 * BLKA, BLKA), 0)
    rr = lax.broadcasted_iota(jnp.int32, (4, 4 * BLKA, BLKA), 1)
    pp = lax.broadcasted_iota(jnp.int32, (4, 4 * BLKA, BLKA), 2)
    pcat[...] = jnp.where(rr == 4 * pp + tt, 1.0, 0.0)
    t2 = lax.broadcasted_iota(jnp.int32, (4, PK, D), 0)
    cc = lax.broadcasted_iota(jnp.int32, (4, PK, D), 1)
    jj = lax.broadcasted_iota(jnp.int32, (4, PK, D), 2)
    szcat[...] = jnp.where(cc == D * t2 + jj, 1.0, 0.0)


_weights = pl.pallas_call(
    _weights_body,
    out_shape=[jax.ShapeDtypeStruct((2 * D, PK), jnp.float32)] * 4 + [
        jax.ShapeDtypeStruct((8, PK), jnp.float32),
        jax.ShapeDtypeStruct((1, PK), jnp.float32),
        jax.ShapeDtypeStruct((PK, 2 * D), jnp.float32),
        jax.ShapeDtypeStruct((PK, 2 * D), jnp.float32),
        jax.ShapeDtypeStruct((PK, PK), jnp.float32),
        jax.ShapeDtypeStruct((D, D), jnp.float32),
        jax.ShapeDtypeStruct((D, D), jnp.float32),
        jax.ShapeDtypeStruct((Y, D), jnp.float32),
        jax.ShapeDtypeStruct((4, 4 * BLKA, BLKA), jnp.float32),
        jax.ShapeDtypeStruct((4, PK, D), jnp.float32)],
)


def _prologue_body(x, enc_pk, b_pk, jd, sel_a, sel_b, c0a, c0b, c1a, c1b):
    ipl = jnp.dot(x[...], enc_pk[...],
                  preferred_element_type=jnp.float32) + b_pk[...]
    jdv = jd[...]
    t = _l2n_pk(ipl, jdv)
    c0 = _l2n_pk(t, jdv)
    c1 = _l2n_pk(c0, jdv)
    c0a[...] = jnp.dot(c0, sel_a[...], preferred_element_type=jnp.float32)
    c0b[...] = jnp.dot(c0, sel_b[...], preferred_element_type=jnp.float32)
    c1a[...] = jnp.dot(c1, sel_a[...], preferred_element_type=jnp.float32)
    c1b[...] = jnp.dot(c1, sel_b[...], preferred_element_type=jnp.float32)


_prologue = pl.pallas_call(
    _prologue_body,
    grid=(NBP,),
    in_specs=[pl.BlockSpec((BLKP, 8), lambda i: (i, 0)),
              pl.BlockSpec((8, PK), lambda i: (0, 0)),
              pl.BlockSpec((1, PK), lambda i: (0, 0)),
              pl.BlockSpec((PK, PK), lambda i: (0, 0)),
              pl.BlockSpec((PK, 2 * D), lambda i: (0, 0)),
              pl.BlockSpec((PK, 2 * D), lambda i: (0, 0))],
    out_specs=[pl.BlockSpec((BLKP, 2 * D), lambda i: (i, 0))] * 4,
    out_shape=[jax.ShapeDtypeStruct((NP4, 2 * D), jnp.float32)] * 4,
)


def _update_body(pa, pb, ca, cb, sgp, w1a_pk, w1b_pk, w2a_pk, w2b_pk,
                 jd, sel_a, sel_b, w1, w2, yc, oa, ob, ynew, acc_a, acc_b):
    i = pl.program_id(0)
    cav, cbv = ca[...], cb[...]
    z = (jnp.dot(pa[...], w1a_pk[...], preferred_element_type=jnp.float32) +
         jnp.dot(pb[...], w1b_pk[...], preferred_element_type=jnp.float32) +
         jnp.dot(cav, w2a_pk[...], preferred_element_type=jnp.float32) +
         jnp.dot(cbv, w2b_pk[...], preferred_element_type=jnp.float32))
    zn = _l2n_pk(jnp.maximum(z, 0.0), jd[...])
    oa[...] = jnp.dot(zn, sel_a[...], preferred_element_type=jnp.float32)
    ob[...] = jnp.dot(zn, sel_b[...], preferred_element_type=jnp.float32)

    @pl.when(i == 0)
    def _():
        acc_a[...] = jnp.zeros((Y, HALF), jnp.float32)
        acc_b[...] = jnp.zeros((Y, HALF), jnp.float32)

    sgv = sgp[0]
    gi = lax.broadcasted_iota(jnp.int32, (Y, BLKP), 0)
    for t in range(4):
        mt = (gi == sgv[t:t + 1, :]).astype(jnp.float32)
        acc_a[...] += jnp.dot(mt, cav[:, t * HALF:(t + 1) * HALF],
                              preferred_element_type=jnp.float32)
        acc_b[...] += jnp.dot(mt, cbv[:, t * HALF:(t + 1) * HALF],
                              preferred_element_type=jnp.float32)

    @pl.when(i == NBP - 1)
    def _():
        w1v, w2v = w1[...], w2[...]
        zy = (jnp.dot(acc_a[...], w1v[0:HALF, :],
                      preferred_element_type=jnp.float32) +
              jnp.dot(acc_b[...], w1v[HALF:D, :],
                      preferred_element_type=jnp.float32) +
              jnp.dot(yc[...], w2v, preferred_element_type=jnp.float32))
        ynew[...] = _l2n(jnp.maximum(zy, 0.0))


_update = pl.pallas_call(
    _update_body,
    grid=(NBP,),
    in_specs=[pl.BlockSpec((BLKP, 2 * D), lambda i: (i, 0))] * 4 + [
        pl.BlockSpec((1, 4, BLKP), lambda i: (i, 0, 0)),
        pl.BlockSpec((2 * D, PK), lambda i: (0, 0)),
        pl.BlockSpec((2 * D, PK), lambda i: (0, 0)),
        pl.BlockSpec((2 * D, PK), lambda i: (0, 0)),
        pl.BlockSpec((2 * D, PK), lambda i: (0, 0)),
        pl.BlockSpec((PK, PK), lambda i: (0, 0)),
        pl.BlockSpec((PK, 2 * D), lambda i: (0, 0)),
        pl.BlockSpec((PK, 2 * D), lambda i: (0, 0)),
        pl.BlockSpec((D, D), lambda i: (0, 0)),
        pl.BlockSpec((D, D), lambda i: (0, 0)),
        pl.BlockSpec((Y, D), lambda i: (0, 0))],
    out_specs=[pl.BlockSpec((BLKP, 2 * D), lambda i: (i, 0)),
               pl.BlockSpec((BLKP, 2 * D), lambda i: (i, 0)),
               pl.BlockSpec((Y, D), lambda i: (0, 0))],
    out_shape=[jax.ShapeDtypeStruct((NP4, 2 * D), jnp.float32),
               jax.ShapeDtypeStruct((NP4, 2 * D), jnp.float32),
               jax.ShapeDtypeStruct((Y, D), jnp.float32)],
    scratch_shapes=[pltpu.VMEM((Y, HALF), jnp.float32),
                    pltpu.VMEM((Y, HALF), jnp.float32)],
)


def _update_last_body(pa, pb, ca, cb, sgp, w1a_pk, w1b_pk, w2a_pk, w2b_pk,
                      jd, pcat, szcat, w1, w2, yc, out, acc_a, acc_b):
    i = pl.program_id(0)
    cav, cbv = ca[...], cb[...]

    @pl.when(i == 0)
    def _():
        acc_a[...] = jnp.zeros((Y, HALF), jnp.float32)
        acc_b[...] = jnp.zeros((Y, HALF), jnp.float32)

    sgv = sgp[0]
    gi = lax.broadcasted_iota(jnp.int32, (Y, BLKA), 0)
    for t in range(4):
        mt = (gi == sgv[t:t + 1, :]).astype(jnp.float32)
        acc_a[...] += jnp.dot(mt, cav[:, t * HALF:(t + 1) * HALF],
                              preferred_element_type=jnp.float32)
        acc_b[...] += jnp.dot(mt, cbv[:, t * HALF:(t + 1) * HALF],
                              preferred_element_type=jnp.float32)

    z = (jnp.dot(pa[...], w1a_pk[...], preferred_element_type=jnp.float32) +
         jnp.dot(pb[...], w1b_pk[...], preferred_element_type=jnp.float32) +
         jnp.dot(cav, w2a_pk[...], preferred_element_type=jnp.float32) +
         jnp.dot(cbv, w2b_pk[...], preferred_element_type=jnp.float32))
    zn = _l2n_pk(jnp.maximum(z, 0.0), jd[...])
    rows = jnp.zeros((4 * BLKA, D), jnp.float32)
    for t in range(4):
        rows = rows + jnp.dot(
            pcat[t], jnp.dot(zn, szcat[t], preferred_element_type=jnp.float32),
            preferred_element_type=jnp.float32)

    w1v, w2v = w1[...], w2[...]
    zy = (jnp.dot(acc_a[...], w1v[0:HALF, :],
                  preferred_element_type=jnp.float32) +
          jnp.dot(acc_b[...], w1v[HALF:D, :],
                  preferred_element_type=jnp.float32) +
          jnp.dot(yc[...], w2v, preferred_element_type=jnp.float32))
    ynv = _l2n(jnp.maximum(zy, 0.0))
    ypad = jnp.concatenate(
        [jnp.zeros((N % (4 * BLKA), D), jnp.float32), ynv,
         jnp.zeros((4 * BLKA - N % (4 * BLKA) - Y, D), jnp.float32)], axis=0)
    grow = (i * 4 * BLKA +
            lax.broadcasted_iota(jnp.int32, (4 * BLKA, 1), 0))
    out[...] = jnp.where(grow < N, rows, ypad)


_update_last = pl.pallas_call(
    _update_last_body,
    grid=(NBA,),
    in_specs=[pl.BlockSpec((BLKA, 2 * D), lambda i: (i, 0))] * 4 + [
        pl.BlockSpec((1, 4, BLKA), lambda i: (i, 0, 0)),
        pl.BlockSpec((2 * D, PK), lambda i: (0, 0)),
        pl.BlockSpec((2 * D, PK), lambda i: (0, 0)),
        pl.BlockSpec((2 * D, PK), lambda i: (0, 0)),
        pl.BlockSpec((2 * D, PK), lambda i: (0, 0)),
        pl.BlockSpec((PK, PK), lambda i: (0, 0)),
        pl.BlockSpec((4, 4 * BLKA, BLKA), lambda i: (0, 0, 0)),
        pl.BlockSpec((4, PK, D), lambda i: (0, 0, 0)),
        pl.BlockSpec((D, D), lambda i: (0, 0)),
        pl.BlockSpec((D, D), lambda i: (0, 0)),
        pl.BlockSpec((Y, D), lambda i: (0, 0))],
    out_specs=pl.BlockSpec((4 * BLKA, D), lambda i: (i, 0)),
    out_shape=jax.ShapeDtypeStruct((N + Y, D), jnp.float32),
    scratch_shapes=[pltpu.VMEM((Y, HALF), jnp.float32),
                    pltpu.VMEM((Y, HALF), jnp.float32)],
)


def kernel(node_input, n2n_index_l0, n2n_value_l0, n2n_index_l1, n2n_value_l1,
           subg_row, subg_value, action_select, aux_input, adj_dummy,
           v_adj_dummy, enc_W, enc_b, w_n2l, p_conv, p_conv2, p_conv3):
    xpk = jnp.zeros((NP4, 8), jnp.float32).at[:N // 4].set(
        node_input[1].reshape(N // 4, 8))
    sg = jnp.full((NPAD,), Y, jnp.int32).at[:N].set(subg_row)
    sgp = sg.reshape(NBP, BLKP, 4).transpose(0, 2, 1)
    sgp64 = sg.reshape(NBA, BLKA, 4).transpose(0, 2, 1)
    rcs = [jnp.stack([idx[0].reshape(NCH, CHUNK),
                      idx[1].reshape(NCH, CHUNK)], axis=1)
           for idx in (n2n_index_l0, n2n_index_l1)]

    (w1a_pk, w1b_pk, w2a_pk, w2b_pk, enc_pk, b_pk, sel_a, sel_b, jd,
     w1, w2, y0, pcat, szcat) = _weights(
        p_conv, p_conv2, p_conv3, w_n2l, enc_W, enc_b.reshape(1, D))
    c0a, c0b, c1a, c1b = _prologue(xpk, enc_pk, b_pk, jd, sel_a, sel_b)

    outs = []
    for l in range(2):
        ca, cb = (c0a, c0b) if l == 0 else (c1a, c1b)
        y = y0
        for it in range(3):
            pa, pb = _get_spmm()(rcs[l], ca.reshape(NPAD, HALF),
                                 cb.reshape(NPAD, HALF))
            pa = pa.reshape(NP4, 2 * D)
            pb = pb.reshape(NP4, 2 * D)
            if it < 2:
                ca, cb, y = _update(pa, pb, ca, cb, sgp, w1a_pk, w1b_pk,
                                    w2a_pk, w2b_pk, jd, sel_a, sel_b,
                                    w1, w2, y)
            else:
                outs.append(_update_last(pa, pb, ca, cb, sgp64,
                                         w1a_pk, w1b_pk, w2a_pk, w2b_pk,
                                         jd, pcat, szcat, w1, w2, y))
    return jnp.stack(outs)


# speedup vs baseline: 1.0552x; 1.0552x over previous
import functools

import jax
import jax.numpy as jnp
from jax import lax
from jax.experimental import pallas as pl
from jax.experimental.pallas import tpu as pltpu
from jax.experimental.pallas import tpu_sc as plsc

N = 50000
E = 800000
D = 64
Y = 64
HALF = 32

NPAD = 50176
NP4 = NPAD // 4
BLKP = 256
NBP = NP4 // BLKP
PK = 4 * D

CHUNK = 128
NCH = E // CHUNK
NTILES = 16
RING = 6
NGRP = -(-((NCH // NTILES) + 1) // RING)
ROWS_PER_TILE = NPAD // NTILES
GROWS = RING * CHUNK

_EPS = 1e-12


def _spmm_body(rc, cur_a, cur_b, pool_a, pool_b,
               accum, ibuf, gbuf, isem, gsem, ssem):
    c = lax.axis_index("c")
    s = lax.axis_index("s")
    cnt = 390 + jnp.where(s < 10, 1, 0)
    start = s * 390 + jnp.minimum(s, 10)
    base = s * ROWS_PER_TILE

    def run_half(cur_hbm, pool_hbm):
        z16 = jnp.zeros((16,), jnp.float32)

        def zbody(i, carry):
            gbuf[i, 0:16] = z16
            gbuf[i, 16:32] = z16
            return carry

        lax.fori_loop(0, GROWS, zbody, 0)
        for t in range(ROWS_PER_TILE // GROWS):
            pltpu.sync_copy(gbuf, accum.at[pl.ds(base + t * GROWS, GROWS)])
        rem = ROWS_PER_TILE % GROWS
        pltpu.sync_copy(gbuf.at[pl.ds(0, rem)],
                        accum.at[pl.ds(base + (ROWS_PER_TILE // GROWS) * GROWS,
                                       rem)])
        plsc.subcore_barrier()

        for b in range(RING):
            pltpu.async_copy(rc.at[start + b], ibuf.at[0, b], isem)

        def body(g, carry):
            par = lax.rem(g, 2)
            nxt = 1 - par

            @pl.when(g > 0)
            def _():
                for b in range(RING):
                    pltpu.make_async_copy(
                        gbuf.at[pl.ds(b * CHUNK, CHUNK)],
                        accum.at[ibuf.at[nxt, b, 0]], ssem).wait()

            for b in range(RING):
                @pl.when(g * RING + b < cnt)
                def _(b=b):
                    pltpu.make_async_copy(rc.at[0], ibuf.at[par, b],
                                          isem).wait()
                    pltpu.async_copy(cur_hbm.at[ibuf.at[par, b, 1]],
                                     gbuf.at[pl.ds(b * CHUNK, CHUNK)], gsem)

            for b in range(RING):
                @pl.when((g + 1) * RING + b < cnt)
                def _(b=b):
                    pltpu.async_copy(rc.at[start + (g + 1) * RING + b],
                                     ibuf.at[nxt, b], isem)

            for b in range(RING):
                @pl.when(g * RING + b < cnt)
                def _(b=b):
                    pltpu.make_async_copy(
                        cur_hbm.at[ibuf.at[par, b, 1]],
                        gbuf.at[pl.ds(b * CHUNK, CHUNK)], gsem).wait()
                    pltpu.async_copy(gbuf.at[pl.ds(b * CHUNK, CHUNK)],
                                     accum.at[ibuf.at[par, b, 0]], ssem,
                                     add=True)
            return carry

        lax.fori_loop(0, NGRP, body, 0)

        lastpar = (NGRP - 1) % 2
        for b in range(RING):
            @pl.when((NGRP - 1) * RING + b < cnt)
            def _(b=b):
                pltpu.make_async_copy(
                    gbuf.at[pl.ds(b * CHUNK, CHUNK)],
                    accum.at[ibuf.at[lastpar, b, 0]], ssem).wait()

        plsc.subcore_barrier()
        pltpu.sync_copy(accum.at[pl.ds(base, ROWS_PER_TILE)],
                        pool_hbm.at[pl.ds(base, ROWS_PER_TILE)])

    @pl.when(c == 0)
    def _():
        run_half(cur_a, pool_a)

    @pl.when(c == 1)
    def _():
        run_half(cur_b, pool_b)


@functools.cache
def _get_spmm():
    return pl.kernel(
        _spmm_body,
        out_type=[jax.ShapeDtypeStruct((NPAD, HALF), jnp.float32),
                  jax.ShapeDtypeStruct((NPAD, HALF), jnp.float32)],
        mesh=plsc.VectorSubcoreMesh(core_axis_name="c", subcore_axis_name="s",
                                    num_cores=2, num_subcores=NTILES),
        scratch_types=[
            pltpu.VMEM_SHARED((NPAD, HALF), jnp.float32),
            pltpu.VMEM((2, RING, 2, CHUNK), jnp.int32),
            pltpu.VMEM((GROWS, HALF), jnp.float32),
            pltpu.SemaphoreType.DMA,
            pltpu.SemaphoreType.DMA,
            pltpu.SemaphoreType.DMA,
        ],
        compiler_params=pltpu.CompilerParams(use_tc_tiling_on_sc=False),
    )


def _bd4(w, r, c):
    t4 = jnp.concatenate([jnp.concatenate([w] * 4, axis=1)] * 4, axis=0)
    ri = lax.broadcasted_iota(jnp.int32, (4 * r, 4 * c), 0)
    ci = lax.broadcasted_iota(jnp.int32, (4 * r, 4 * c), 1)
    return jnp.where(ri // r == ci // c, t4, 0.0)


def _l2n(x):
    n = jnp.sqrt(jnp.sum(x * x, axis=1, keepdims=True))
    return x / jnp.maximum(n, _EPS)


def _l2n_pk(x, jd):
    nsq = jnp.dot(x * x, jd, preferred_element_type=jnp.float32)
    return x / jnp.maximum(jnp.sqrt(nsq), _EPS)


def _weights_body(pc, pc2, pc3, wn, ew, eb,
                  w1a_pk, w1b_pk, w2a_pk, w2b_pk, enc_pk, b_pk,
                  sel_a, sel_b, jd, w1o, w2o, y0):
    w1 = jnp.dot(pc[...], pc3[0:D, :], preferred_element_type=jnp.float32)
    w2 = jnp.dot(pc2[...], pc3[D:2 * D, :],
                 preferred_element_type=jnp.float32)
    w1o[...] = w1
    w2o[...] = w2
    w1a_pk[...] = _bd4(w1[0:HALF, :], HALF, D)
    w1b_pk[...] = _bd4(w1[HALF:D, :], HALF, D)
    w2a_pk[...] = _bd4(w2[0:HALF, :], HALF, D)
    w2b_pk[...] = _bd4(w2[HALF:D, :], HALF, D)
    enc_pk[...] = _bd4(ew[...], 2, D)
    b_pk[...] = jnp.concatenate([eb[...]] * 4, axis=1)
    ri = lax.broadcasted_iota(jnp.int32, (PK, 2 * D), 0)
    ci = lax.broadcasted_iota(jnp.int32, (PK, 2 * D), 1)
    hit = ri // D == ci // HALF
    sel_a[...] = jnp.where(hit & (ri % D == ci % HALF), 1.0, 0.0)
    sel_b[...] = jnp.where(hit & (ri % D == ci % HALF + HALF), 1.0, 0.0)
    ri2 = lax.broadcasted_iota(jnp.int32, (PK, PK), 0)
    ci2 = lax.broadcasted_iota(jnp.int32, (PK, PK), 1)
    jd[...] = jnp.where(ri2 // D == ci2 // D, 1.0, 0.0)
    r = jnp.maximum(wn[0:1, :] + wn[1:2, :], 0.0)
    y0[...] = _l2n(jnp.broadcast_to(r, (Y, D)))


_weights = pl.pallas_call(
    _weights_body,
    out_shape=[jax.ShapeDtypeStruct((2 * D, PK), jnp.float32)] * 4 + [
        jax.ShapeDtypeStruct((8, PK), jnp.float32),
        jax.ShapeDtypeStruct((1, PK), jnp.float32),
        jax.ShapeDtypeStruct((PK, 2 * D), jnp.float32),
        jax.ShapeDtypeStruct((PK, 2 * D), jnp.float32),
        jax.ShapeDtypeStruct((PK, PK), jnp.float32),
        jax.ShapeDtypeStruct((D, D), jnp.float32),
        jax.ShapeDtypeStruct((D, D), jnp.float32),
        jax.ShapeDtypeStruct((Y, D), jnp.float32)],
)


def _prologue_body(x, enc_pk, b_pk, jd, sel_a, sel_b, c0a, c0b, c1a, c1b):
    ipl = jnp.dot(x[...], enc_pk[...],
                  preferred_element_type=jnp.float32) + b_pk[...]
    jdv = jd[...]
    t = _l2n_pk(ipl, jdv)
    c0 = _l2n_pk(t, jdv)
    c1 = _l2n_pk(c0, jdv)
    c0a[...] = jnp.dot(c0, sel_a[...], preferred_element_type=jnp.float32)
    c0b[...] = jnp.dot(c0, sel_b[...], preferred_element_type=jnp.float32)
    c1a[...] = jnp.dot(c1, sel_a[...], preferred_element_type=jnp.float32)
    c1b[...] = jnp.dot(c1, sel_b[...], preferred_element_type=jnp.float32)


_prologue = pl.pallas_call(
    _prologue_body,
    grid=(NBP,),
    in_specs=[pl.BlockSpec((BLKP, 8), lambda i: (i, 0)),
              pl.BlockSpec((8, PK), lambda i: (0, 0)),
              pl.BlockSpec((1, PK), lambda i: (0, 0)),
              pl.BlockSpec((PK, PK), lambda i: (0, 0)),
              pl.BlockSpec((PK, 2 * D), lambda i: (0, 0)),
              pl.BlockSpec((PK, 2 * D), lambda i: (0, 0))],
    out_specs=[pl.BlockSpec((BLKP, 2 * D), lambda i: (i, 0))] * 4,
    out_shape=[jax.ShapeDtypeStruct((NP4, 2 * D), jnp.float32)] * 4,
)


def _update_body(pa, pb, ca, cb, sgp, w1a_pk, w1b_pk, w2a_pk, w2b_pk,
                 jd, sel_a, sel_b, w1, w2, yc, oa, ob, ynew, acc_a, acc_b):
    i = pl.program_id(0)
    cav, cbv = ca[...], cb[...]
    z = (jnp.dot(pa[...], w1a_pk[...], preferred_element_type=jnp.float32) +
         jnp.dot(pb[...], w1b_pk[...], preferred_element_type=jnp.float32) +
         jnp.dot(cav, w2a_pk[...], preferred_element_type=jnp.float32) +
         jnp.dot(cbv, w2b_pk[...], preferred_element_type=jnp.float32))
    zn = _l2n_pk(jnp.maximum(z, 0.0), jd[...])
    oa[...] = jnp.dot(zn, sel_a[...], preferred_element_type=jnp.float32)
    ob[...] = jnp.dot(zn, sel_b[...], preferred_element_type=jnp.float32)

    @pl.when(i == 0)
    def _():
        acc_a[...] = jnp.zeros((Y, HALF), jnp.float32)
        acc_b[...] = jnp.zeros((Y, HALF), jnp.float32)

    sgv = sgp[0]
    gi = lax.broadcasted_iota(jnp.int32, (Y, BLKP), 0)
    for t in range(4):
        mt = (gi == sgv[t:t + 1, :]).astype(jnp.float32)
        acc_a[...] += jnp.dot(mt, cav[:, t * HALF:(t + 1) * HALF],
                              preferred_element_type=jnp.float32)
        acc_b[...] += jnp.dot(mt, cbv[:, t * HALF:(t + 1) * HALF],
                              preferred_element_type=jnp.float32)

    @pl.when(i == NBP - 1)
    def _():
        w1v, w2v = w1[...], w2[...]
        zy = (jnp.dot(acc_a[...], w1v[0:HALF, :],
                      preferred_element_type=jnp.float32) +
              jnp.dot(acc_b[...], w1v[HALF:D, :],
                      preferred_element_type=jnp.float32) +
              jnp.dot(yc[...], w2v, preferred_element_type=jnp.float32))
        ynew[...] = _l2n(jnp.maximum(zy, 0.0))


_update = pl.pallas_call(
    _update_body,
    grid=(NBP,),
    in_specs=[pl.BlockSpec((BLKP, 2 * D), lambda i: (i, 0))] * 4 + [
        pl.BlockSpec((1, 4, BLKP), lambda i: (i, 0, 0)),
        pl.BlockSpec((2 * D, PK), lambda i: (0, 0)),
        pl.BlockSpec((2 * D, PK), lambda i: (0, 0)),
        pl.BlockSpec((2 * D, PK), lambda i: (0, 0)),
        pl.BlockSpec((2 * D, PK), lambda i: (0, 0)),
        pl.BlockSpec((PK, PK), lambda i: (0, 0)),
        pl.BlockSpec((PK, 2 * D), lambda i: (0, 0)),
        pl.BlockSpec((PK, 2 * D), lambda i: (0, 0)),
        pl.BlockSpec((D, D), lambda i: (0, 0)),
        pl.BlockSpec((D, D), lambda i: (0, 0)),
        pl.BlockSpec((Y, D), lambda i: (0, 0))],
    out_specs=[pl.BlockSpec((BLKP, 2 * D), lambda i: (i, 0)),
               pl.BlockSpec((BLKP, 2 * D), lambda i: (i, 0)),
               pl.BlockSpec((Y, D), lambda i: (0, 0))],
    out_shape=[jax.ShapeDtypeStruct((NP4, 2 * D), jnp.float32),
               jax.ShapeDtypeStruct((NP4, 2 * D), jnp.float32),
               jax.ShapeDtypeStruct((Y, D), jnp.float32)],
    scratch_shapes=[pltpu.VMEM((Y, HALF), jnp.float32),
                    pltpu.VMEM((Y, HALF), jnp.float32)],
)


def kernel(node_input, n2n_index_l0, n2n_value_l0, n2n_index_l1, n2n_value_l1,
           subg_row, subg_value, action_select, aux_input, adj_dummy,
           v_adj_dummy, enc_W, enc_b, w_n2l, p_conv, p_conv2, p_conv3):
    xpk = jnp.zeros((NP4, 8), jnp.float32).at[:N // 4].set(
        node_input[1].reshape(N // 4, 8))
    sg = jnp.full((NPAD,), Y, jnp.int32).at[:N].set(subg_row)
    sgp = sg.reshape(NBP, BLKP, 4).transpose(0, 2, 1)
    rcs = [jnp.stack([idx[0].reshape(NCH, CHUNK),
                      idx[1].reshape(NCH, CHUNK)], axis=1)
           for idx in (n2n_index_l0, n2n_index_l1)]

    (w1a_pk, w1b_pk, w2a_pk, w2b_pk, enc_pk, b_pk, sel_a, sel_b, jd,
     w1, w2, y0) = _weights(
        p_conv, p_conv2, p_conv3, w_n2l, enc_W, enc_b.reshape(1, D))
    c0a, c0b, c1a, c1b = _prologue(xpk, enc_pk, b_pk, jd, sel_a, sel_b)

    outs = []
    for l in range(2):
        ca, cb = (c0a, c0b) if l == 0 else (c1a, c1b)
        y = y0
        for it in range(3):
            pa, pb = _get_spmm()(rcs[l], ca.reshape(NPAD, HALF),
                                 cb.reshape(NPAD, HALF))
            ca, cb, y = _update(pa.reshape(NP4, 2 * D),
                                pb.reshape(NP4, 2 * D), ca, cb,
                                sgp, w1a_pk, w1b_pk, w2a_pk, w2b_pk,
                                jd, sel_a, sel_b, w1, w2, y)
        cur = jnp.concatenate([ca.reshape(NPAD, HALF)[:N],
                               cb.reshape(NPAD, HALF)[:N]], axis=1)
        outs.append(jnp.concatenate([cur, y], axis=0))
    return jnp.stack(outs)
